# Initial kernel scaffold; baseline (speedup 1.0000x reference)
import jax
import jax.numpy as jnp
from jax.experimental import pallas as pl


def kernel(x, weights, indices):
    raise NotImplementedError("write your pallas kernel here")


# speedup vs baseline: 3.4643x; 3.4643x over previous
import jax
import jax.numpy as jnp
from jax import lax
from jax.experimental import pallas as pl
from jax.experimental.pallas import tpu as pltpu
from jax.experimental.pallas import tpu_sc as plsc

NCHIP = 4
SEQ = 2048
HID = 2048
TOPK = 2
NEXP = 16
MAXD = 1536
MLEN = 8
P = NCHIP * SEQ * TOPK
NCORE = 2
NSUB = 16
NW = NCORE * NSUB
CHUNK = P // NW
NGRP = CHUNK // 16
OUT_ROWS = NEXP * MAXD
STAG_ROWS = OUT_ROWS + 16
STAG_W = 128
ROWS_PER_W = OUT_ROWS // NW
NWIN = ROWS_PER_W // 16

_mesh = plsc.VectorSubcoreMesh(core_axis_name="c", subcore_axis_name="s")
_cparams = pltpu.CompilerParams(needs_layout_passes=False)


def _wid():
    return lax.axis_index("c") * NSUB + lax.axis_index("s")


def _iota():
    return lax.iota(jnp.int32, 16)


def _rank_kernel(idx_hbm, hist_hbm, lrank_hbm, idx_v, lr_v, rc_v, sb_v):
    wid = _wid()
    base = wid * CHUNK
    pltpu.sync_copy(idx_hbm.at[pl.ds(base, CHUNK)], idx_v)
    zeros = jnp.zeros((16,), jnp.int32)
    rc_v[...] = zeros
    sb_v[pl.ds(0, 16)] = jnp.full((16,), -1, jnp.int32)
    sb_v[pl.ds(32, 16)] = jnp.full((16,), -2, jnp.int32)

    @pl.loop(0, NGRP)
    def _grp(g):
        v = idx_v[pl.ds(g * 16, 16)]
        sb_v[pl.ds(16, 16)] = v
        occ = jnp.zeros((16,), jnp.int32)
        later = jnp.zeros((16,), jnp.int32)
        for s in range(1, 16):
            occ = occ + (sb_v[pl.ds(16 - s, 16)] == v).astype(jnp.int32)
            later = later + (sb_v[pl.ds(16 + s, 16)] == v).astype(jnp.int32)
        run = plsc.load_gather(rc_v, [v])
        lr = run + occ
        lr_v[pl.ds(g * 16, 16)] = lr
        plsc.store_scatter(rc_v, [v], lr + 1, mask=later == 0)

    pltpu.sync_copy(lr_v, lrank_hbm.at[pl.ds(base, CHUNK)])
    pltpu.sync_copy(rc_v, hist_hbm.at[pl.ds(wid * 16, 16)])


def _route_kernel(idx_hbm, lrank_hbm, w_hbm, hist_hbm,
                  stag_hbm, cnt_hbm,
                  idx_v, lr_v, w_v, hist_v, off_v, tot_v, data_v, idxb_v):
    wid = _wid()
    base = wid * CHUNK
    pltpu.sync_copy(idx_hbm.at[pl.ds(base, CHUNK)], idx_v)
    pltpu.sync_copy(lrank_hbm.at[pl.ds(base, CHUNK)], lr_v)
    pltpu.sync_copy(w_hbm.at[pl.ds(base, CHUNK)], w_v)
    pltpu.sync_copy(hist_hbm, hist_v)
    iota = _iota()
    zeros = jnp.zeros((16,), jnp.int32)

    off_v[...] = zeros

    @pl.loop(0, NW)
    def _off(w):
        @pl.when(w < wid)
        def _():
            off_v[...] = off_v[...] + hist_v[pl.ds(w * 16, 16)]

    @pl.when(wid == 0)
    def _counters():
        tot_v[...] = zeros

        @pl.loop(0, NW)
        def _tot(w):
            tot_v[...] = tot_v[...] + hist_v[pl.ds(w * 16, 16)]

        pltpu.sync_copy(tot_v, cnt_hbm)

    @pl.loop(0, 128)
    def _z(r):
        @pl.loop(0, STAG_W // 16)
        def _zc(c):
            plsc.store_scatter(
                data_v, [jnp.full((16,), r, jnp.int32), c * 16 + iota], zeros)

    @pl.loop(0, NGRP)
    def _grp(g):
        b = lax.rem(g, 8)
        j = lax.div(g, 8)
        v = idx_v[pl.ds(g * 16, 16)]
        lr = lr_v[pl.ds(g * 16, 16)]
        wf = w_v[pl.ds(g * 16, 16)]
        offe = plsc.load_gather(off_v, [v])
        rank = lr + offe
        valid = rank < MAXD
        p_vec = base + g * 16 + iota
        out_row = jnp.where(valid, v * MAXD + rank, OUT_ROWS + iota)
        tok = lax.shift_right_logical(p_vec, 1)
        chip = lax.shift_right_logical(p_vec, 12)
        token = jnp.bitwise_and(tok, SEQ - 1)
        topk = jnp.bitwise_and(p_vec, 1)
        w_int = wf.astype(jnp.int32)
        row_idx = b * 16 + iota

        def put(col, vals):
            plsc.store_scatter(data_v, [row_idx, jnp.full((16,), col, jnp.int32)], vals)

        put(0, chip)
        put(1, token)
        put(2, topk)
        put(3, v)
        put(4, w_int)
        put(8, tok)
        plsc.store_scatter(idxb_v, [jnp.full((16,), j, jnp.int32), b * 16 + iota], out_row)

        @pl.when(b == 7)
        def _flush():
            pltpu.sync_copy(data_v, stag_hbm.at[idxb_v.at[j]])


def _dispatch_kernel(x_hbm, stag_hbm, hist_hbm,
                     disp_hbm, meta_hbm,
                     hist_v, tot_v, stag_v, tok_v, meta_v, rows_v, zero_v):
    wid = _wid()
    e = lax.div(wid, 2)
    half = lax.rem(wid, 2)
    rstart = e * MAXD + half * ROWS_PER_W
    iota = _iota()
    zeros = jnp.zeros((16,), jnp.int32)
    zf = jnp.zeros((16,), jnp.float32)

    pltpu.sync_copy(hist_hbm, hist_v)
    tot_v[...] = zeros

    @pl.loop(0, NW)
    def _tot(w):
        tot_v[...] = tot_v[...] + hist_v[pl.ds(w * 16, 16)]

    tot = tot_v[...]
    cnt = jnp.sum(jnp.where(iota == e, tot, 0))
    cnt = jnp.minimum(cnt, MAXD)
    nfill = jnp.clip(cnt - half * ROWS_PER_W, 0, ROWS_PER_W)

    @pl.loop(0, 16)
    def _zr(r):
        @pl.loop(0, HID // 16)
        def _zc(c):
            plsc.store_scatter(zero_v, [jnp.full((16,), r, jnp.int32), c * 16 + iota], zf)

    col = jnp.bitwise_and(iota, 7)
    rowp = lax.shift_right_logical(iota, 3)
    col8 = jnp.full((16,), 8, jnp.int32)

    @pl.loop(0, NWIN)
    def _win(i):
        rbase = rstart + i * 16
        nv = jnp.clip(nfill - i * 16, 0, 16)
        nv_v = jnp.full((16,), nv, jnp.int32)
        pltpu.sync_copy(stag_hbm.at[pl.ds(rbase, 16)], stag_v)
        tok_raw = plsc.load_gather(stag_v, [iota, col8])
        tok_v[...] = jnp.where(iota < nv_v, tok_raw, 0)
        for j in range(8):
            ridx = 2 * j + rowp
            mv = plsc.load_gather(stag_v, [ridx, col])
            mv = jnp.where(ridx < nv_v, mv, -1)
            plsc.store_scatter(meta_v, [ridx, col], mv)
        pltpu.sync_copy(meta_v, meta_hbm.at[pl.ds(rbase, 16)])

        @pl.when(nv > 0)
        def _gather():
            pltpu.sync_copy(x_hbm.at[tok_v], rows_v)
            pltpu.sync_copy(rows_v, disp_hbm.at[pl.ds(rbase, 16)])

            @pl.when(nv < 16)
            def _tail():
                @pl.loop(0, 16)
                def _row(r):
                    @pl.when(r >= nv)
                    def _():
                        pltpu.sync_copy(zero_v.at[0], disp_hbm.at[rbase + r])

        @pl.when(nv == 0)
        def _zero():
            pltpu.sync_copy(zero_v, disp_hbm.at[pl.ds(rbase, 16)])


def kernel(x, weights, indices):
    x_flat = x.reshape(NCHIP * SEQ, HID)
    idx_flat = indices.reshape(P).astype(jnp.int32)
    w_flat = weights.reshape(P)

    rank_call = pl.kernel(
        _rank_kernel,
        out_type=(
            jax.ShapeDtypeStruct((NW * 16,), jnp.int32),
            jax.ShapeDtypeStruct((P,), jnp.int32),
        ),
        mesh=_mesh,
        compiler_params=_cparams,
        scratch_types=[
            pltpu.VMEM((CHUNK,), jnp.int32),
            pltpu.VMEM((CHUNK,), jnp.int32),
            pltpu.VMEM((16,), jnp.int32),
            pltpu.VMEM((48,), jnp.int32),
        ],
    )
    hist, lrank = rank_call(idx_flat)

    route_call = pl.kernel(
        _route_kernel,
        out_type=(
            jax.ShapeDtypeStruct((STAG_ROWS, STAG_W), jnp.int32),
            jax.ShapeDtypeStruct((16,), jnp.int32),
        ),
        mesh=_mesh,
        compiler_params=_cparams,
        scratch_types=[
            pltpu.VMEM((CHUNK,), jnp.int32),
            pltpu.VMEM((CHUNK,), jnp.int32),
            pltpu.VMEM((CHUNK,), jnp.float32),
            pltpu.VMEM((NW * 16,), jnp.int32),
            pltpu.VMEM((16,), jnp.int32),
            pltpu.VMEM((16,), jnp.int32),
            pltpu.VMEM((128, STAG_W), jnp.int32),
            pltpu.VMEM((4, 128), jnp.int32),
        ],
    )
    staging, counters = route_call(idx_flat, lrank, w_flat, hist)

    dispatch_call = pl.kernel(
        _dispatch_kernel,
        out_type=(
            jax.ShapeDtypeStruct((OUT_ROWS, HID), jnp.float32),
            jax.ShapeDtypeStruct((OUT_ROWS, MLEN), jnp.int32),
        ),
        mesh=_mesh,
        compiler_params=_cparams,
        scratch_types=[
            pltpu.VMEM((NW * 16,), jnp.int32),
            pltpu.VMEM((16,), jnp.int32),
            pltpu.VMEM((16, STAG_W), jnp.int32),
            pltpu.VMEM((16,), jnp.int32),
            pltpu.VMEM((16, MLEN), jnp.int32),
            pltpu.VMEM((16, HID), jnp.float32),
            pltpu.VMEM((16, HID), jnp.float32),
        ],
    )
    dispatched, metadata = dispatch_call(x_flat, staging, hist)

    return (
        dispatched.reshape(NCHIP, NEXP // NCHIP, MAXD, HID),
        metadata.reshape(NCHIP, NEXP // NCHIP, MAXD, MLEN),
        counters.reshape(NCHIP, NEXP // NCHIP),
    )
